# Initial kernel scaffold; baseline (speedup 1.0000x reference)
import jax
import jax.numpy as jnp
from jax.experimental import pallas as pl


def kernel(x1, x2, params):
    raise NotImplementedError("write your pallas kernel here")


# speedup vs baseline: 1.7384x; 1.7384x over previous
import jax
import jax.numpy as jnp
from jax.experimental import pallas as pl
from jax.experimental.pallas import tpu as pltpu


def _dot(a, b):
    return jnp.dot(a, b, preferred_element_type=jnp.float32)


def _dot_rhs_t(a, b):
    return jax.lax.dot_general(a, b, (((1,), (1,)), ((), ())),
                               preferred_element_type=jnp.float32)


def _branch_update(h, Wa_ref, ba_ref, Wb_ref, bb_ref, WcT_ref, bcT_ref,
                   m_ref, s_ref, acc_ref):
    gated = jnp.tanh(_dot(h, Wa_ref[...]) + ba_ref[...]) * \
        jax.nn.sigmoid(_dot(h, Wb_ref[...]) + bb_ref[...])
    sc = _dot_rhs_t(WcT_ref[...], gated) + bcT_ref[...]
    m_old = m_ref[...]
    m_new = jnp.maximum(m_old, jnp.max(sc, axis=1, keepdims=True))
    alpha = jnp.exp(m_old - m_new)
    p = jnp.exp(sc - m_new)
    m_ref[...] = m_new
    s_ref[...] = s_ref[...] * alpha + jnp.sum(p, axis=1, keepdims=True)
    acc_ref[...] = acc_ref[...] * alpha + _dot(p, h)


def _stream_body(x1_ref, x2_ref, Wp_ref, bp_ref, Wvf_ref, bvf_ref,
                 Wva_ref, bva_ref, Wvb_ref, bvb_ref, WvcT_ref, bvcT_ref,
                 Wuf_ref, buf_ref, Wua_ref, bua_ref, Wub_ref, bub_ref,
                 WucT_ref, bucT_ref,
                 M1_ref, M2_ref,
                 Wfold_ref, bfold_ref,
                 mv_ref, sv_ref, accv_ref, mu_ref, su_ref, accu_ref):
    i = pl.program_id(0)
    n = pl.num_programs(0)

    @pl.when(i == 0)
    def _init():
        Wfold_ref[...] = _dot(Wp_ref[...], Wvf_ref[...])
        bfold_ref[...] = _dot(bp_ref[...], Wvf_ref[...]) + bvf_ref[...]
        for r in (mv_ref, mu_ref):
            r[...] = jnp.full_like(r[...], -jnp.inf)
        for r in (sv_ref, accv_ref, su_ref, accu_ref):
            r[...] = jnp.zeros_like(r[...])

    hv = jnp.maximum(_dot(x1_ref[...], Wfold_ref[...]) + bfold_ref[...], 0.0)
    _branch_update(hv, Wva_ref, bva_ref, Wvb_ref, bvb_ref, WvcT_ref, bvcT_ref,
                   mv_ref, sv_ref, accv_ref)

    hu = jnp.maximum(_dot(x2_ref[...], Wuf_ref[...]) + buf_ref[...], 0.0)
    _branch_update(hu, Wua_ref, bua_ref, Wub_ref, bub_ref, WucT_ref, bucT_ref,
                   mu_ref, su_ref, accu_ref)

    @pl.when(i == n - 1)
    def _fin():
        M1_ref[...] = accv_ref[...] / sv_ref[...]
        M2_ref[...] = accu_ref[...] / su_ref[...]


def _tail_body(M1_ref, M2_ref,
               W1a_ref, b1a_ref, W1b_ref, b1b_ref,
               W3a_ref, b3a_ref, W3b_ref, b3b_ref,
               W2a_ref, b2a_ref, W2b_ref, b2b_ref,
               Wop_ref, bop_ref, Wg1_ref, bg1_ref, Wg2_ref, bg2_ref,
               Wc_ref, bc_ref, out_ref):
    M1 = M1_ref[...]
    M2 = M2_ref[...]
    cat = jnp.concatenate([M1, M2], axis=1)
    relu = lambda v: jnp.maximum(v, 0.0)
    e1 = relu(_dot(relu(_dot(M1, W1a_ref[...]) + b1a_ref[...]),
                   W1b_ref[...]) + b1b_ref[...])
    e3 = relu(_dot(relu(_dot(M2, W3a_ref[...]) + b3a_ref[...]),
                   W3b_ref[...]) + b3b_ref[...])
    e2 = _dot(relu(_dot(relu(_dot(cat, W2a_ref[...]) + b2a_ref[...]),
                        W2b_ref[...]) + b2b_ref[...]),
              Wop_ref[...]) + bop_ref[...]
    glog = _dot(relu(_dot(cat, Wg1_ref[...]) + bg1_ref[...]),
                Wg2_ref[...]) + bg2_ref[...]
    g = jax.nn.softmax(glog, axis=1)
    fused = g[:, 0:1] * e1 + g[:, 1:2] * e2 + g[:, 2:3] * e3
    logits = jnp.sum(fused * Wc_ref[...], axis=1, keepdims=True)
    out_ref[...] = logits.reshape(1, 2) + bc_ref[...]


def _pick_chunk(n):
    best = None
    for c in range(min(n, 1024), 0, -1):
        if n % c == 0:
            if c % 8 == 0:
                return c
            if best is None:
                best = c
    return best


def kernel(x1, x2, params):
    (Wp, bp, Wvf, bvf, Wva, bva, Wvb, bvb, Wvc, bvc,
     Wuf, buf, Wua, bua, Wub, bub, Wuc, buc,
     W1a, b1a, W1b, b1b, W3a, b3a, W3b, b3b,
     W2a, b2a, W2b, b2b, Wop, bop,
     Wg1, bg1, Wg2, bg2, Wc, bc) = params

    N = x1.shape[0]
    C = _pick_chunk(N)
    G = N // C
    f32 = jnp.float32

    row = lambda v: v.reshape(1, -1)
    const2 = lambda a: pl.BlockSpec(a.shape, lambda i: (0, 0))

    stream_in = [
        x1, x2, Wp, row(bp), Wvf, row(bvf),
        Wva, row(bva), Wvb, row(bvb), Wvc.T, bvc.reshape(2, 1),
        Wuf, row(buf), Wua, row(bua), Wub, row(bub), Wuc.T, buc.reshape(2, 1),
    ]
    in_specs = [
        pl.BlockSpec((C, x1.shape[1]), lambda i: (i, 0)),
        pl.BlockSpec((C, x2.shape[1]), lambda i: (i, 0)),
    ] + [const2(a) for a in stream_in[2:]]

    M1, M2 = pl.pallas_call(
        _stream_body,
        grid=(G,),
        in_specs=in_specs,
        out_specs=[const2(jnp.zeros((2, 512))) for _ in range(2)],
        out_shape=[jax.ShapeDtypeStruct((2, 512), f32) for _ in range(2)],
        scratch_shapes=[
            pltpu.VMEM((Wp.shape[0], Wvf.shape[1]), f32),
            pltpu.VMEM((1, Wvf.shape[1]), f32),
            pltpu.VMEM((2, 1), f32), pltpu.VMEM((2, 1), f32),
            pltpu.VMEM((2, 512), f32),
            pltpu.VMEM((2, 1), f32), pltpu.VMEM((2, 1), f32),
            pltpu.VMEM((2, 512), f32),
        ],
        compiler_params=pltpu.CompilerParams(
            dimension_semantics=("arbitrary",)),
    )(*stream_in)

    tail_in = [
        M1, M2, W1a, row(b1a), W1b, row(b1b), W3a, row(b3a), W3b, row(b3b),
        W2a, row(b2a), W2b, row(b2b), Wop, row(bop),
        Wg1, row(bg1), Wg2, row(bg2), Wc, row(bc),
    ]
    out = pl.pallas_call(
        _tail_body,
        out_shape=jax.ShapeDtypeStruct((1, 2), f32),
    )(*tail_in)
    return out
